# baseline (speedup 1.0000x reference)
import functools

import jax
import jax.numpy as jnp
from jax import lax
from jax.experimental import pallas as pl
from jax.experimental.pallas import tpu as pltpu
from jax.experimental.pallas import tpu_sc as plsc

_RC2 = 9.0
_E0 = (1.0 / 3.0) ** 12 - (1.0 / 3.0) ** 6
_N = 8_000_000
_NC = 2
_NS = 16
_NW = _NC * _NS
_RPW = _N // _NW
_C = 10_000
_NCH = _RPW // _C
_GRP = _C // 16

_mesh = plsc.VectorSubcoreMesh(core_axis_name="c", subcore_axis_name="s")


@functools.partial(
    pl.kernel,
    mesh=_mesh,
    compiler_params=pltpu.CompilerParams(needs_layout_passes=False),
    out_type=jax.ShapeDtypeStruct((_NW, 16), jnp.float32),
    scratch_types=[
        pltpu.VMEM((_C * 3,), jnp.float32),
        pltpu.VMEM((_C * 3,), jnp.float32),
        pltpu.VMEM((_C,), jnp.int32),
        pltpu.VMEM((_C,), jnp.int32),
        pltpu.VMEM((16,), jnp.float32),
        pltpu.SemaphoreType.DMA,
        pltpu.SemaphoreType.DMA,
        pltpu.SemaphoreType.DMA,
        pltpu.SemaphoreType.DMA,
    ],
)
def _lj_sc(b_hbm, r_hbm, out_hbm, r0, r1, b0, b1, ov, sr0, sr1, sb0, sb1):
    wid = lax.axis_index("s") * _NC + lax.axis_index("c")
    base = wid * _RPW
    rbufs, bbufs = (r0, r1), (b0, b1)
    rsems, bsems = (sr0, sr1), (sb0, sb1)

    def start(i, par):
        st = base + i * _C
        rc = pltpu.async_copy(r_hbm.at[pl.ds(st * 3, _C * 3)], rbufs[par], rsems[par])
        bc = pltpu.async_copy(b_hbm.at[pl.ds(st, _C)], bbufs[par], bsems[par])
        return rc, bc

    copies = [None, None]
    copies[0] = start(0, 0)
    iota3 = lax.iota(jnp.int32, 16) * 3
    acc = jnp.zeros((16,), jnp.float32)
    for i in range(_NCH):
        par = i % 2
        if i + 1 < _NCH:
            copies[(i + 1) % 2] = start(i + 1, (i + 1) % 2)
        rc, bc = copies[par]
        rc.wait()
        bc.wait()
        rbuf, bbuf = rbufs[par], bbufs[par]

        def body(g, acc):
            ix = g * 48 + iota3
            x = plsc.load_gather(rbuf, [ix])
            y = plsc.load_gather(rbuf, [ix + 1])
            z = plsc.load_gather(rbuf, [ix + 2])
            bv = bbuf[pl.ds(g * 16, 16)]
            d2 = x * x + y * y + z * z
            m = (bv == 0) & (d2 < _RC2)
            sd = jnp.where(m, d2, 1.0)
            t = 1.0 / sd
            t3 = t * t * t
            e = t3 * t3 - t3 - _E0
            return acc + jnp.where(m, e, 0.0)

        acc = lax.fori_loop(0, _GRP, body, acc)
    ov[...] = acc
    pltpu.sync_copy(ov, out_hbm.at[wid])


def kernel(a, b, r):
    partials = _lj_sc(b, r.reshape(-1))
    e = 4.0 * jnp.sum(partials)
    return jnp.where(a == 0, e / 2.0, jnp.asarray(0.0, dtype=jnp.float32))


# speedup vs baseline: 317.7286x; 317.7286x over previous
import functools

import jax
import jax.numpy as jnp
from jax import lax
from jax.experimental import pallas as pl
from jax.experimental.pallas import tpu as pltpu
from jax.experimental.pallas import tpu_sc as plsc

_SC = 1.0 / 729.0
_SCM1 = _SC - 1.0
_N = 8_000_000
_NBLK = _N // 128
_NW = 32
_BB = 672
_TCB = 12096
_TSTEPS = _TCB // _BB
_SCB = _NBLK - _TCB
_BPW = _SCB // _NW
_EXTRA = _SCB - _BPW * _NW
_CB = 63
_NCH = _BPW // _CB
_GRP = _CB * 8

_mesh = plsc.VectorSubcoreMesh(core_axis_name="c", subcore_axis_name="s")


@functools.partial(
    pl.kernel,
    mesh=_mesh,
    out_type=jax.ShapeDtypeStruct((_NW, 16), jnp.float32),
    scratch_types=[
        pltpu.VMEM((_CB, 3, 128), jnp.float32),
        pltpu.VMEM((_CB, 3, 128), jnp.float32),
        pltpu.VMEM((_CB, 3, 128), jnp.float32),
        pltpu.VMEM((_CB * 128,), jnp.int32),
        pltpu.VMEM((_CB * 128,), jnp.int32),
        pltpu.VMEM((_CB * 128,), jnp.int32),
        pltpu.VMEM((1, 3, 128), jnp.float32),
        pltpu.VMEM((128,), jnp.int32),
        pltpu.VMEM((16,), jnp.float32),
        pltpu.SemaphoreType.DMA,
        pltpu.SemaphoreType.DMA,
        pltpu.SemaphoreType.DMA,
        pltpu.SemaphoreType.DMA,
        pltpu.SemaphoreType.DMA,
        pltpu.SemaphoreType.DMA,
        pltpu.SemaphoreType.DMA,
    ],
)
def _lj_sc(
    b_hbm, rt_hbm, out_hbm,
    r0, r1, r2, b0, b1, b2, rt_t, bt_t, ov,
    sr0, sr1, sr2, sb0, sb1, sb2, st,
):
    cid = lax.axis_index("c")
    sid = lax.axis_index("s")
    wid = sid * 2 + cid
    base_blk = _TCB + wid * _BPW + jnp.minimum(wid, _EXTRA)
    has_tail = wid < _EXTRA
    rbufs, bbufs = (r0, r1, r2), (b0, b1, b2)
    rsems, bsems = (sr0, sr1, sr2), (sb0, sb1, sb2)

    def start(i, par):
        blk = base_blk + i * _CB
        rc = pltpu.async_copy(rt_hbm.at[pl.ds(blk, _CB)], rbufs[par], rsems[par])
        bc = pltpu.async_copy(
            b_hbm.at[pl.ds(blk * 128, _CB * 128)], bbufs[par], bsems[par]
        )
        return rc, bc

    ones16 = jnp.ones((16,), jnp.int32)
    zeros16 = jnp.zeros((16,), jnp.float32)
    for k in range(8):
        bt_t[pl.ds(k * 16, 16)] = ones16
        for c in range(3):
            rt_t[0, c, pl.ds(k * 16, 16)] = zeros16

    copies = [None, None, None]
    copies[0] = start(0, 0)
    copies[1] = start(1, 1)

    @pl.when(has_tail)
    def _():
        tail_blk = base_blk + _BPW
        pltpu.async_copy(rt_hbm.at[pl.ds(tail_blk, 1)], rt_t, st)
        pltpu.async_copy(b_hbm.at[pl.ds(tail_blk * 128, 128)], bt_t, st)

    acc = jnp.zeros((16,), jnp.float32)

    def group(rbuf, bbuf, bl, k, t, acc):
        sl = pl.ds(k * 16, 16)
        x = rbuf[bl, 0, sl]
        y = rbuf[bl, 1, sl]
        z = rbuf[bl, 2, sl]
        bv = bbuf[pl.ds(t * 16, 16)]
        d2 = x * x + y * y + z * z + bv.astype(jnp.float32) * 16.0
        t3 = 1.0 / (d2 * d2 * d2)
        u = jnp.maximum(t3, _SC)
        return acc + (u - _SC) * (u + _SCM1)

    for i in range(_NCH):
        par = i % 3
        if i + 2 < _NCH:
            copies[(i + 2) % 3] = start(i + 2, (i + 2) % 3)
        rc, bc = copies[par]
        rc.wait()
        bc.wait()
        rbuf, bbuf = rbufs[par], bbufs[par]

        def body(t, acc, rbuf=rbuf, bbuf=bbuf):
            return group(rbuf, bbuf, t >> 3, t & 7, t, acc)

        acc = plsc.parallel_loop(0, _GRP, unroll=8, carry=acc)(body)

    @pl.when(has_tail)
    def _():
        tail_blk = base_blk + _BPW
        pltpu.make_async_copy(rt_hbm.at[pl.ds(tail_blk, 1)], rt_t, st).wait()
        pltpu.make_async_copy(
            b_hbm.at[pl.ds(tail_blk * 128, 128)], bt_t, st
        ).wait()

    def tail_body(k, a):
        return group(rt_t, bt_t, 0, k, k, a)

    acc = lax.fori_loop(0, 8, tail_body, acc)
    ov[...] = acc
    pltpu.sync_copy(ov, out_hbm.at[wid])


def _tc_body(rt_ref, b_ref, o_ref):
    @pl.when(pl.program_id(0) == 0)
    def _():
        o_ref[...] = jnp.zeros_like(o_ref)

    x = rt_ref[:, 0, :]
    y = rt_ref[:, 1, :]
    z = rt_ref[:, 2, :]
    bv = b_ref[...]
    d2 = x * x + y * y + z * z + bv.astype(jnp.float32) * 16.0
    t3 = 1.0 / (d2 * d2 * d2)
    u = jnp.maximum(t3, _SC)
    e = (u - _SC) * (u + _SCM1)
    o_ref[...] += jnp.sum(e.reshape(_BB // 8, 8, 128), axis=0)


_lj_tc = pl.pallas_call(
    _tc_body,
    grid=(_TSTEPS,),
    in_specs=[
        pl.BlockSpec((_BB, 3, 128), lambda i: (i, 0, 0)),
        pl.BlockSpec((_BB, 128), lambda i: (i, 0)),
    ],
    out_specs=pl.BlockSpec((8, 128), lambda i: (0, 0)),
    out_shape=jax.ShapeDtypeStruct((8, 128), jnp.float32),
)


def kernel(a, b, r):
    rt = r.reshape(_NBLK, 128, 3).transpose(0, 2, 1)
    partials = _lj_sc(b, rt)
    tc_part = _lj_tc(rt, b.reshape(_NBLK, 128))
    e = 4.0 * (jnp.sum(partials) + jnp.sum(tc_part))
    return jnp.where(a == 0, e / 2.0, jnp.asarray(0.0, dtype=jnp.float32))
